# Initial kernel scaffold; baseline (speedup 1.0000x reference)
import jax
import jax.numpy as jnp
from jax.experimental import pallas as pl


def kernel(Q, K, V, l_q1, l_k1, l_q2, l_k2):
    raise NotImplementedError("write your pallas kernel here")


# speedup vs baseline: 4.5302x; 4.5302x over previous
import functools
import math

import jax
import jax.numpy as jnp
import numpy as np
from jax import lax
from jax.experimental import pallas as pl
from jax.experimental.pallas import tpu as pltpu

_LAMBDA_INIT = 0.8 - 0.6 * math.exp(-0.3 * 2048)


def _round_up(x, m):
    return (x + m - 1) // m * m


@functools.lru_cache(maxsize=None)
def _dft_weights(L):
    F = L // 2 + 1
    Fp = _round_up(F, 128)
    t = np.arange(L)
    f = np.arange(Fp)
    ang = 2.0 * np.pi * np.outer(f, t) / L
    valid = (f[:, None] <= L // 2)
    Fc = (np.cos(ang) * valid).astype(np.float32)
    Fs = (-np.sin(ang) * valid).astype(np.float32)
    w = np.where((f == 0) | (f == L // 2), 1.0, 2.0) * (f <= L // 2) / L
    ICc = np.ascontiguousarray((np.cos(ang) * w[:, None]).T).astype(np.float32)
    ICi = np.ascontiguousarray((-np.sin(ang) * w[:, None]).T).astype(np.float32)
    return Fc, Fs, ICc, ICi


def _stage1_body(NF, HE, B, g_ref, q_ref, k_ref, fc_ref, fs_ref, icc_ref,
                 ici_ref, out_ref, corr_ref):
    b = pl.program_id(0)
    h2 = pl.program_id(1)
    fi = pl.program_id(2)

    @pl.when(fi == 0)
    def _():
        corr_ref[...] = jnp.zeros_like(corr_ref)

    q = q_ref[0]
    k = k_ref[0]
    fc = fc_ref[...]
    fs = fs_ref[...]
    qr = jnp.dot(fc, q, preferred_element_type=jnp.float32, precision=lax.Precision.HIGHEST)
    qi = jnp.dot(fs, q, preferred_element_type=jnp.float32, precision=lax.Precision.HIGHEST)
    kr = jnp.dot(fc, k, preferred_element_type=jnp.float32, precision=lax.Precision.HIGHEST)
    ki = jnp.dot(fs, k, preferred_element_type=jnp.float32, precision=lax.Precision.HIGHEST)
    rr = qr * kr + qi * ki
    ri = qi * kr - qr * ki
    corr_ref[...] += (jnp.dot(icc_ref[...], rr, preferred_element_type=jnp.float32, precision=lax.Precision.HIGHEST)
                      + jnp.dot(ici_ref[...], ri, preferred_element_type=jnp.float32, precision=lax.Precision.HIGHEST))

    @pl.when(fi == NF - 1)
    def _():
        c = corr_ref[...]
        m = jnp.max(c, axis=0, keepdims=True)
        e = jnp.exp(c - m)
        s = jnp.sum(e, axis=0, keepdims=True)
        col = jnp.sum(e / s, axis=1, keepdims=True)
        contrib = col * (g_ref[0, 0] / HE)
        L = c.shape[0]
        lane = lax.broadcasted_iota(jnp.int32, (L, B), 1)
        cur = out_ref[...]
        base = jnp.where(h2 > 0, cur, jnp.zeros_like(cur))
        out_ref[...] = jnp.where(lane == b, base + contrib, cur)


def _stage2_body(L, B, K7, KP, mv_ref, d_ref, w_ref):
    x = mv_ref[...]
    row = lax.broadcasted_iota(jnp.int32, (L, B), 0)
    krow = lax.broadcasted_iota(jnp.int32, (KP, B), 0)
    dmat = jnp.zeros((KP, B), jnp.int32)
    wmat = jnp.full((KP, B), -jnp.inf, jnp.float32)
    v0 = None
    for i in range(K7):
        m = jnp.max(x, axis=0, keepdims=True)
        idx = jnp.min(jnp.where(x >= m, row, L), axis=0, keepdims=True)
        dmat = jnp.where(krow == i, idx, dmat)
        wmat = jnp.where(krow == i, m, wmat)
        if v0 is None:
            v0 = m
        x = jnp.where(row == idx, -jnp.inf, x)
    e = jnp.exp(wmat - v0)
    w = e / jnp.sum(e, axis=0, keepdims=True)
    d_ref[...] = dmat
    w_ref[...] = w


def _stage3_body(L, K7, d_ref, w_ref, v_ref, o_ref, v2_ref):
    b = pl.program_id(0)
    v = v_ref[0, 0]
    v2_ref[pl.ds(0, L), :] = v
    v2_ref[pl.ds(L, L), :] = v
    acc = jnp.zeros(v.shape, jnp.float32)
    for i in range(K7):
        d = d_ref[i, b]
        w = w_ref[i, b]
        acc = acc + w * v2_ref[pl.ds(d, L), :]
    o_ref[0, 0] = acc


def kernel(Q, K, V, l_q1, l_k1, l_q2, l_k2):
    B, H, L, E = Q.shape
    HE = H * E
    K7 = max(1, int(math.log(L)))
    KP = _round_up(K7, 8)
    HEb = min(512, HE)
    NH2 = HE // HEb

    l1 = jnp.exp(jnp.sum(l_q1 * l_k1))
    l2 = jnp.exp(jnp.sum(l_q2 * l_k2))
    g = 1.0 - (l1 - l2 + _LAMBDA_INIT)
    g_arr = jnp.reshape(g, (1, 1)).astype(jnp.float32)

    Qt = jnp.transpose(Q, (0, 2, 1, 3)).reshape(B, L, HE)
    Kt = jnp.transpose(K, (0, 2, 1, 3)).reshape(B, L, HE)

    Fc, Fs, ICc, ICi = _dft_weights(L)
    Fp = Fc.shape[0]
    FB = 128
    NF = Fp // FB
    Fc = jnp.asarray(Fc)
    Fs = jnp.asarray(Fs)
    ICc = jnp.asarray(ICc)
    ICi = jnp.asarray(ICi)

    mv = pl.pallas_call(
        functools.partial(_stage1_body, NF, HE, B),
        grid=(B, NH2, NF),
        in_specs=[
            pl.BlockSpec(memory_space=pltpu.SMEM),
            pl.BlockSpec((1, L, HEb), lambda b, h, f: (b, 0, h)),
            pl.BlockSpec((1, L, HEb), lambda b, h, f: (b, 0, h)),
            pl.BlockSpec((FB, L), lambda b, h, f: (f, 0)),
            pl.BlockSpec((FB, L), lambda b, h, f: (f, 0)),
            pl.BlockSpec((L, FB), lambda b, h, f: (0, f)),
            pl.BlockSpec((L, FB), lambda b, h, f: (0, f)),
        ],
        out_specs=pl.BlockSpec((L, B), lambda b, h, f: (0, 0)),
        out_shape=jax.ShapeDtypeStruct((L, B), jnp.float32),
        scratch_shapes=[pltpu.VMEM((L, HEb), jnp.float32)],
    )(g_arr, Qt, Kt, Fc, Fs, ICc, ICi)

    delays, weights = pl.pallas_call(
        functools.partial(_stage2_body, L, B, K7, KP),
        in_specs=[pl.BlockSpec((L, B), lambda: (0, 0))],
        out_specs=[
            pl.BlockSpec((KP, B), lambda: (0, 0)),
            pl.BlockSpec((KP, B), lambda: (0, 0)),
        ],
        out_shape=[
            jax.ShapeDtypeStruct((KP, B), jnp.int32),
            jax.ShapeDtypeStruct((KP, B), jnp.float32),
        ],
    )(mv)

    out = pl.pallas_call(
        functools.partial(_stage3_body, L, K7),
        grid=(B, H),
        in_specs=[
            pl.BlockSpec((KP, B), lambda b, h: (0, 0), memory_space=pltpu.SMEM),
            pl.BlockSpec((KP, B), lambda b, h: (0, 0), memory_space=pltpu.SMEM),
            pl.BlockSpec((1, 1, L, E), lambda b, h: (b, h, 0, 0)),
        ],
        out_specs=pl.BlockSpec((1, 1, L, E), lambda b, h: (b, h, 0, 0)),
        out_shape=jax.ShapeDtypeStruct((B, H, L, E), jnp.float32),
        scratch_shapes=[pltpu.VMEM((2 * L, E), jnp.float32)],
    )(delays, weights, V)

    return out


# speedup vs baseline: 5.3948x; 1.1909x over previous
import functools
import math

import jax
import jax.numpy as jnp
import numpy as np
from jax import lax
from jax.experimental import pallas as pl
from jax.experimental.pallas import tpu as pltpu

_LAMBDA_INIT = 0.8 - 0.6 * math.exp(-0.3 * 2048)


def _round_up(x, m):
    return (x + m - 1) // m * m


@functools.lru_cache(maxsize=None)
def _dft_weights(L):
    F = L // 2 + 1
    Fp = _round_up(F, 256)
    t = np.arange(L)
    f = np.arange(Fp)
    ang = 2.0 * np.pi * np.outer(f, t) / L
    valid = (f[:, None] <= L // 2)
    Fc = (np.cos(ang) * valid).astype(np.float32)
    Fs = (-np.sin(ang) * valid).astype(np.float32)
    w = np.where((f == 0) | (f == L // 2), 1.0, 2.0) * (f <= L // 2) / L
    ICc = np.ascontiguousarray((np.cos(ang) * w[:, None]).T).astype(np.float32)
    ICi = np.ascontiguousarray((-np.sin(ang) * w[:, None]).T).astype(np.float32)
    return Fc, Fs, ICc, ICi


def _stage1_body(NF, HE, B, g_ref, q_ref, k_ref, fc_ref, fs_ref, icc_ref,
                 ici_ref, out_ref, corr_ref):
    b = pl.program_id(0)
    h2 = pl.program_id(1)
    fi = pl.program_id(2)

    @pl.when(fi == 0)
    def _():
        corr_ref[...] = jnp.zeros_like(corr_ref)

    q = q_ref[0]
    k = k_ref[0]
    fc = fc_ref[...]
    fs = fs_ref[...]
    qr = jnp.dot(fc, q, preferred_element_type=jnp.float32, precision=lax.Precision.HIGHEST)
    qi = jnp.dot(fs, q, preferred_element_type=jnp.float32, precision=lax.Precision.HIGHEST)
    kr = jnp.dot(fc, k, preferred_element_type=jnp.float32, precision=lax.Precision.HIGHEST)
    ki = jnp.dot(fs, k, preferred_element_type=jnp.float32, precision=lax.Precision.HIGHEST)
    rr = qr * kr + qi * ki
    ri = qi * kr - qr * ki
    corr_ref[...] += (jnp.dot(icc_ref[...], rr, preferred_element_type=jnp.float32, precision=lax.Precision.HIGHEST)
                      + jnp.dot(ici_ref[...], ri, preferred_element_type=jnp.float32, precision=lax.Precision.HIGHEST))

    @pl.when(fi == NF - 1)
    def _():
        c = corr_ref[...]
        m = jnp.max(c, axis=0, keepdims=True)
        e = jnp.exp(c - m)
        s = jnp.sum(e, axis=0, keepdims=True)
        col = jnp.sum(e / s, axis=1, keepdims=True)
        contrib = col * (g_ref[0, 0] / HE)
        L = c.shape[0]
        lane = lax.broadcasted_iota(jnp.int32, (L, B), 1)
        cur = out_ref[...]
        base = jnp.where(h2 > 0, cur, jnp.zeros_like(cur))
        out_ref[...] = jnp.where(lane == b, base + contrib, cur)


def _stage2_body(L, B, K7, KP, mv_ref, d_ref, w_ref):
    x = mv_ref[...]
    row = lax.broadcasted_iota(jnp.int32, (L, B), 0)
    krow = lax.broadcasted_iota(jnp.int32, (KP, B), 0)
    dmat = jnp.zeros((KP, B), jnp.int32)
    wmat = jnp.full((KP, B), -jnp.inf, jnp.float32)
    v0 = None
    for i in range(K7):
        m = jnp.max(x, axis=0, keepdims=True)
        idx = jnp.min(jnp.where(x >= m, row, L), axis=0, keepdims=True)
        dmat = jnp.where(krow == i, idx, dmat)
        wmat = jnp.where(krow == i, m, wmat)
        if v0 is None:
            v0 = m
        x = jnp.where(row == idx, -jnp.inf, x)
    e = jnp.exp(wmat - v0)
    w = e / jnp.sum(e, axis=0, keepdims=True)
    d_ref[...] = dmat
    w_ref[...] = w


def _stage3_body(L, K7, d_ref, w_ref, v_ref, o_ref, v2_ref):
    b = pl.program_id(0)
    v = v_ref[0, 0]
    v2_ref[pl.ds(0, L), :] = v
    v2_ref[pl.ds(L, L), :] = v
    acc = jnp.zeros(v.shape, jnp.float32)
    for i in range(K7):
        d = d_ref[i, b]
        w = w_ref[i, b]
        acc = acc + w * v2_ref[pl.ds(d, L), :]
    o_ref[0, 0] = acc


def kernel(Q, K, V, l_q1, l_k1, l_q2, l_k2):
    B, H, L, E = Q.shape
    HE = H * E
    K7 = max(1, int(math.log(L)))
    KP = _round_up(K7, 8)
    HEb = min(512, HE)
    NH2 = HE // HEb

    l1 = jnp.exp(jnp.sum(l_q1 * l_k1))
    l2 = jnp.exp(jnp.sum(l_q2 * l_k2))
    g = 1.0 - (l1 - l2 + _LAMBDA_INIT)
    g_arr = jnp.reshape(g, (1, 1)).astype(jnp.float32)

    Qt = jnp.transpose(Q, (0, 2, 1, 3)).reshape(B, L, HE)
    Kt = jnp.transpose(K, (0, 2, 1, 3)).reshape(B, L, HE)

    Fc, Fs, ICc, ICi = _dft_weights(L)
    Fp = Fc.shape[0]
    FB = 256
    NF = Fp // FB
    Fc = jnp.asarray(Fc)
    Fs = jnp.asarray(Fs)
    ICc = jnp.asarray(ICc)
    ICi = jnp.asarray(ICi)

    mv = pl.pallas_call(
        functools.partial(_stage1_body, NF, HE, B),
        grid=(B, NH2, NF),
        in_specs=[
            pl.BlockSpec(memory_space=pltpu.SMEM),
            pl.BlockSpec((1, L, HEb), lambda b, h, f: (b, 0, h)),
            pl.BlockSpec((1, L, HEb), lambda b, h, f: (b, 0, h)),
            pl.BlockSpec((FB, L), lambda b, h, f: (f, 0)),
            pl.BlockSpec((FB, L), lambda b, h, f: (f, 0)),
            pl.BlockSpec((L, FB), lambda b, h, f: (0, f)),
            pl.BlockSpec((L, FB), lambda b, h, f: (0, f)),
        ],
        out_specs=pl.BlockSpec((L, B), lambda b, h, f: (0, 0)),
        out_shape=jax.ShapeDtypeStruct((L, B), jnp.float32),
        scratch_shapes=[pltpu.VMEM((L, HEb), jnp.float32)],
        compiler_params=pltpu.CompilerParams(vmem_limit_bytes=100 * 1024 * 1024),
    )(g_arr, Qt, Kt, Fc, Fs, ICc, ICi)

    delays, weights = pl.pallas_call(
        functools.partial(_stage2_body, L, B, K7, KP),
        in_specs=[pl.BlockSpec((L, B), lambda: (0, 0))],
        out_specs=[
            pl.BlockSpec((KP, B), lambda: (0, 0)),
            pl.BlockSpec((KP, B), lambda: (0, 0)),
        ],
        out_shape=[
            jax.ShapeDtypeStruct((KP, B), jnp.int32),
            jax.ShapeDtypeStruct((KP, B), jnp.float32),
        ],
    )(mv)

    out = pl.pallas_call(
        functools.partial(_stage3_body, L, K7),
        grid=(B, H),
        in_specs=[
            pl.BlockSpec((KP, B), lambda b, h: (0, 0), memory_space=pltpu.SMEM),
            pl.BlockSpec((KP, B), lambda b, h: (0, 0), memory_space=pltpu.SMEM),
            pl.BlockSpec((1, 1, L, E), lambda b, h: (b, h, 0, 0)),
        ],
        out_specs=pl.BlockSpec((1, 1, L, E), lambda b, h: (b, h, 0, 0)),
        out_shape=jax.ShapeDtypeStruct((B, H, L, E), jnp.float32),
        scratch_shapes=[pltpu.VMEM((2 * L, E), jnp.float32)],
    )(delays, weights, V)

    return out


# speedup vs baseline: 8.7425x; 1.6205x over previous
import functools
import math

import jax
import jax.numpy as jnp
import numpy as np
from jax import lax
from jax.experimental import pallas as pl
from jax.experimental.pallas import tpu as pltpu

_LAMBDA_INIT = 0.8 - 0.6 * math.exp(-0.3 * 2048)


def _round_up(x, m):
    return (x + m - 1) // m * m


@functools.lru_cache(maxsize=None)
def _dft_weights(L):
    F = L // 2 + 1
    Fp = _round_up(F, 256)
    t = np.arange(L)
    f = np.arange(Fp)
    ang = 2.0 * np.pi * np.outer(f, t) / L
    valid = (f[:, None] <= L // 2)
    Fc = (np.cos(ang) * valid).astype(np.float32)
    Fs = (-np.sin(ang) * valid).astype(np.float32)
    w = np.where((f == 0) | (f == L // 2), 1.0, 2.0) * (f <= L // 2) / L
    ICc = np.ascontiguousarray((np.cos(ang) * w[:, None]).T).astype(np.float32)
    ICi = np.ascontiguousarray((-np.sin(ang) * w[:, None]).T).astype(np.float32)
    return Fc, Fs, ICc, ICi


def _hi_lo(x):
    hi = x.astype(jnp.bfloat16)
    lo = (x - hi.astype(jnp.float32)).astype(jnp.bfloat16)
    return hi, lo


def _dot3(wh, wl, xh, xl):
    f32 = jnp.float32
    return (jnp.dot(wh, xh, preferred_element_type=f32)
            + jnp.dot(wh, xl, preferred_element_type=f32)
            + jnp.dot(wl, xh, preferred_element_type=f32))


def _stage1_body(NF, HE, B, g_ref, q_ref, k_ref, fch_ref, fcl_ref, fsh_ref,
                 fsl_ref, icch_ref, iccl_ref, icih_ref, icil_ref, out_ref,
                 corr_ref, qh_ref, ql_ref, kh_ref, kl_ref):
    b = pl.program_id(0)
    h2 = pl.program_id(1)
    fi = pl.program_id(2)

    @pl.when(fi == 0)
    def _():
        corr_ref[...] = jnp.zeros_like(corr_ref)
        qh, ql = _hi_lo(q_ref[0])
        qh_ref[...] = qh
        ql_ref[...] = ql
        kh, kl = _hi_lo(k_ref[0])
        kh_ref[...] = kh
        kl_ref[...] = kl

    qh = qh_ref[...]
    ql = ql_ref[...]
    kh = kh_ref[...]
    kl = kl_ref[...]
    qr = _dot3(fch_ref[...], fcl_ref[...], qh, ql)
    qi = _dot3(fsh_ref[...], fsl_ref[...], qh, ql)
    kr = _dot3(fch_ref[...], fcl_ref[...], kh, kl)
    ki = _dot3(fsh_ref[...], fsl_ref[...], kh, kl)
    rr = qr * kr + qi * ki
    ri = qi * kr - qr * ki
    rrh, rrl = _hi_lo(rr)
    rih, ril = _hi_lo(ri)
    corr_ref[...] += (_dot3(icch_ref[...], iccl_ref[...], rrh, rrl)
                      + _dot3(icih_ref[...], icil_ref[...], rih, ril))

    @pl.when(fi == NF - 1)
    def _():
        c = corr_ref[...]
        m = jnp.max(c, axis=0, keepdims=True)
        e = jnp.exp(c - m)
        s = jnp.sum(e, axis=0, keepdims=True)
        col = jnp.sum(e / s, axis=1, keepdims=True)
        contrib = col * (g_ref[0, 0] / HE)
        L = c.shape[0]
        lane = lax.broadcasted_iota(jnp.int32, (L, B), 1)
        cur = out_ref[...]
        base = jnp.where(h2 > 0, cur, jnp.zeros_like(cur))
        out_ref[...] = jnp.where(lane == b, base + contrib, cur)


def _stage2_body(L, B, K7, KP, mv_ref, d_ref, w_ref):
    x = mv_ref[...]
    row = lax.broadcasted_iota(jnp.int32, (L, B), 0)
    krow = lax.broadcasted_iota(jnp.int32, (KP, B), 0)
    dmat = jnp.zeros((KP, B), jnp.int32)
    wmat = jnp.full((KP, B), -jnp.inf, jnp.float32)
    v0 = None
    for i in range(K7):
        m = jnp.max(x, axis=0, keepdims=True)
        idx = jnp.min(jnp.where(x >= m, row, L), axis=0, keepdims=True)
        dmat = jnp.where(krow == i, idx, dmat)
        wmat = jnp.where(krow == i, m, wmat)
        if v0 is None:
            v0 = m
        x = jnp.where(row == idx, -jnp.inf, x)
    e = jnp.exp(wmat - v0)
    w = e / jnp.sum(e, axis=0, keepdims=True)
    d_ref[...] = dmat
    w_ref[...] = w


def _stage3_body(L, K7, d_ref, w_ref, v_ref, o_ref, v2_ref):
    b = pl.program_id(0)
    v = v_ref[0, 0]
    v2_ref[pl.ds(0, L), :] = v
    v2_ref[pl.ds(L, L), :] = v
    acc = jnp.zeros(v.shape, jnp.float32)
    for i in range(K7):
        d = d_ref[i, b]
        w = w_ref[i, b]
        acc = acc + w * v2_ref[pl.ds(d, L), :]
    o_ref[0, 0] = acc


def kernel(Q, K, V, l_q1, l_k1, l_q2, l_k2):
    B, H, L, E = Q.shape
    HE = H * E
    K7 = max(1, int(math.log(L)))
    KP = _round_up(K7, 8)
    HEb = min(512, HE)
    NH2 = HE // HEb

    l1 = jnp.exp(jnp.sum(l_q1 * l_k1))
    l2 = jnp.exp(jnp.sum(l_q2 * l_k2))
    g = 1.0 - (l1 - l2 + _LAMBDA_INIT)
    g_arr = jnp.reshape(g, (1, 1)).astype(jnp.float32)

    Qt = jnp.transpose(Q, (0, 2, 1, 3)).reshape(B, L, HE)
    Kt = jnp.transpose(K, (0, 2, 1, 3)).reshape(B, L, HE)

    Fc, Fs, ICc, ICi = _dft_weights(L)
    Fp = Fc.shape[0]
    FB = 256
    NF = Fp // FB
    fw = []
    for wmat in (Fc, Fs, ICc, ICi):
        hi, lo = _hi_lo(wmat)
        fw.append(jnp.asarray(hi))
        fw.append(jnp.asarray(lo))

    fwd_spec = pl.BlockSpec((FB, L), lambda b, h, f: (f, 0))
    inv_spec = pl.BlockSpec((L, FB), lambda b, h, f: (0, f))
    mv = pl.pallas_call(
        functools.partial(_stage1_body, NF, HE, B),
        grid=(B, NH2, NF),
        in_specs=[
            pl.BlockSpec(memory_space=pltpu.SMEM),
            pl.BlockSpec((1, L, HEb), lambda b, h, f: (b, 0, h)),
            pl.BlockSpec((1, L, HEb), lambda b, h, f: (b, 0, h)),
            fwd_spec, fwd_spec, fwd_spec, fwd_spec,
            inv_spec, inv_spec, inv_spec, inv_spec,
        ],
        out_specs=pl.BlockSpec((L, B), lambda b, h, f: (0, 0)),
        out_shape=jax.ShapeDtypeStruct((L, B), jnp.float32),
        scratch_shapes=[
            pltpu.VMEM((L, HEb), jnp.float32),
            pltpu.VMEM((L, HEb), jnp.bfloat16),
            pltpu.VMEM((L, HEb), jnp.bfloat16),
            pltpu.VMEM((L, HEb), jnp.bfloat16),
            pltpu.VMEM((L, HEb), jnp.bfloat16),
        ],
        compiler_params=pltpu.CompilerParams(vmem_limit_bytes=100 * 1024 * 1024),
    )(g_arr, Qt, Kt, *fw)

    delays, weights = pl.pallas_call(
        functools.partial(_stage2_body, L, B, K7, KP),
        in_specs=[pl.BlockSpec((L, B), lambda: (0, 0))],
        out_specs=[
            pl.BlockSpec((KP, B), lambda: (0, 0)),
            pl.BlockSpec((KP, B), lambda: (0, 0)),
        ],
        out_shape=[
            jax.ShapeDtypeStruct((KP, B), jnp.int32),
            jax.ShapeDtypeStruct((KP, B), jnp.float32),
        ],
    )(mv)

    out = pl.pallas_call(
        functools.partial(_stage3_body, L, K7),
        grid=(B, H),
        in_specs=[
            pl.BlockSpec((KP, B), lambda b, h: (0, 0), memory_space=pltpu.SMEM),
            pl.BlockSpec((KP, B), lambda b, h: (0, 0), memory_space=pltpu.SMEM),
            pl.BlockSpec((1, 1, L, E), lambda b, h: (b, h, 0, 0)),
        ],
        out_specs=pl.BlockSpec((1, 1, L, E), lambda b, h: (b, h, 0, 0)),
        out_shape=jax.ShapeDtypeStruct((B, H, L, E), jnp.float32),
        scratch_shapes=[pltpu.VMEM((2 * L, E), jnp.float32)],
    )(delays, weights, V)

    return out


# speedup vs baseline: 9.6562x; 1.1045x over previous
import functools
import math

import jax
import jax.numpy as jnp
import numpy as np
from jax import lax
from jax.experimental import pallas as pl
from jax.experimental.pallas import tpu as pltpu

_LAMBDA_INIT = 0.8 - 0.6 * math.exp(-0.3 * 2048)


def _round_up(x, m):
    return (x + m - 1) // m * m


@functools.lru_cache(maxsize=None)
def _dft_weights(L):
    F2 = L // 2
    t = np.arange(L)
    f = np.arange(F2)
    ang = 2.0 * np.pi * np.outer(f, t) / L
    Fc = np.cos(ang).astype(np.float32)
    Fs = (-np.sin(ang)).astype(np.float32)
    w = np.where(f == 0, 1.0, 2.0) / L
    ICc = np.ascontiguousarray((np.cos(ang) * w[:, None]).T).astype(np.float32)
    ICi = np.ascontiguousarray((-np.sin(ang) * w[:, None]).T).astype(np.float32)
    return Fc, Fs, ICc, ICi


def _hi_lo(x):
    hi = x.astype(jnp.bfloat16)
    lo = (x - hi.astype(jnp.float32)).astype(jnp.bfloat16)
    return hi, lo


def _dot3(wh, wl, xh, xl):
    f32 = jnp.float32
    return (jnp.dot(wh, xh, preferred_element_type=f32)
            + jnp.dot(wh, xl, preferred_element_type=f32)
            + jnp.dot(wl, xh, preferred_element_type=f32))


def _stage1_body(HE, B, L, g_ref, qh_ref, ql_ref, kh_ref, kl_ref,
                 fch_ref, fcl_ref, fsh_ref, fsl_ref,
                 icch_ref, iccl_ref, icih_ref, icil_ref, out_ref):
    b = pl.program_id(0)
    h2 = pl.program_id(1)

    qh = qh_ref[0]
    ql = ql_ref[0]
    kh = kh_ref[0]
    kl = kl_ref[0]
    qr = _dot3(fch_ref[...], fcl_ref[...], qh, ql)
    qi = _dot3(fsh_ref[...], fsl_ref[...], qh, ql)
    kr = _dot3(fch_ref[...], fcl_ref[...], kh, kl)
    ki = _dot3(fsh_ref[...], fsl_ref[...], kh, kl)
    rr = qr * kr + qi * ki
    ri = qi * kr - qr * ki
    rrh, rrl = _hi_lo(rr)
    rih, ril = _hi_lo(ri)
    c = (_dot3(icch_ref[...], iccl_ref[...], rrh, rrl)
         + _dot3(icih_ref[...], icil_ref[...], rih, ril))

    alt = (1.0 - 2.0 * (lax.broadcasted_iota(jnp.int32, (L, 1), 0) % 2)
           ).astype(jnp.float32)
    qa = (jnp.sum(qh * alt, axis=0, keepdims=True)
          + jnp.sum(ql * alt, axis=0, keepdims=True))
    ka = (jnp.sum(kh * alt, axis=0, keepdims=True)
          + jnp.sum(kl * alt, axis=0, keepdims=True))
    c = c + alt * (qa * ka * (1.0 / L))

    m = jnp.max(c, axis=0, keepdims=True)
    e = jnp.exp(c - m)
    s = jnp.sum(e, axis=0, keepdims=True)
    col = jnp.sum(e / s, axis=1, keepdims=True)
    contrib = col * (g_ref[0, 0] / HE)
    lane = lax.broadcasted_iota(jnp.int32, (L, B), 1)
    cur = out_ref[...]
    base = jnp.where(h2 > 0, cur, jnp.zeros_like(cur))
    out_ref[...] = jnp.where(lane == b, base + contrib, cur)


def _stage2_body(L, B, K7, KP, mv_ref, d_ref, w_ref):
    x = mv_ref[...]
    row = lax.broadcasted_iota(jnp.int32, (L, B), 0)
    krow = lax.broadcasted_iota(jnp.int32, (KP, B), 0)
    dmat = jnp.zeros((KP, B), jnp.int32)
    wmat = jnp.full((KP, B), -jnp.inf, jnp.float32)
    v0 = None
    for i in range(K7):
        m = jnp.max(x, axis=0, keepdims=True)
        idx = jnp.min(jnp.where(x >= m, row, L), axis=0, keepdims=True)
        dmat = jnp.where(krow == i, idx, dmat)
        wmat = jnp.where(krow == i, m, wmat)
        if v0 is None:
            v0 = m
        x = jnp.where(row == idx, -jnp.inf, x)
    e = jnp.exp(wmat - v0)
    w = e / jnp.sum(e, axis=0, keepdims=True)
    d_ref[...] = dmat
    w_ref[...] = w


def _stage3_body(L, K7, d_ref, w_ref, v_ref, o_ref, v2_ref):
    b = pl.program_id(0)
    v = v_ref[0, 0]
    v2_ref[pl.ds(0, L), :] = v
    v2_ref[pl.ds(L, L), :] = v
    acc = jnp.zeros(v.shape, jnp.float32)
    for i in range(K7):
        d = d_ref[i, b]
        w = w_ref[i, b]
        acc = acc + w * v2_ref[pl.ds(d, L), :]
    o_ref[0, 0] = acc


def kernel(Q, K, V, l_q1, l_k1, l_q2, l_k2):
    B, H, L, E = Q.shape
    HE = H * E
    K7 = max(1, int(math.log(L)))
    KP = _round_up(K7, 8)
    HEb = min(256, HE)
    NH2 = HE // HEb

    l1 = jnp.exp(jnp.sum(l_q1 * l_k1))
    l2 = jnp.exp(jnp.sum(l_q2 * l_k2))
    g = 1.0 - (l1 - l2 + _LAMBDA_INIT)
    g_arr = jnp.reshape(g, (1, 1)).astype(jnp.float32)

    Qt = jnp.transpose(Q, (0, 2, 1, 3)).reshape(B, L, HE)
    Kt = jnp.transpose(K, (0, 2, 1, 3)).reshape(B, L, HE)
    Qh, Ql = _hi_lo(Qt)
    Kh, Kl = _hi_lo(Kt)

    Fc, Fs, ICc, ICi = _dft_weights(L)
    F2 = Fc.shape[0]
    fw = []
    for wmat in (Fc, Fs, ICc, ICi):
        hi, lo = _hi_lo(wmat)
        fw.append(jnp.asarray(hi))
        fw.append(jnp.asarray(lo))

    data_spec = pl.BlockSpec((1, L, HEb), lambda b, h: (b, 0, h))
    fwd_spec = pl.BlockSpec((F2, L), lambda b, h: (0, 0))
    inv_spec = pl.BlockSpec((L, F2), lambda b, h: (0, 0))
    mv = pl.pallas_call(
        functools.partial(_stage1_body, HE, B, L),
        grid=(B, NH2),
        in_specs=[
            pl.BlockSpec(memory_space=pltpu.SMEM),
            data_spec, data_spec, data_spec, data_spec,
            fwd_spec, fwd_spec, fwd_spec, fwd_spec,
            inv_spec, inv_spec, inv_spec, inv_spec,
        ],
        out_specs=pl.BlockSpec((L, B), lambda b, h: (0, 0)),
        out_shape=jax.ShapeDtypeStruct((L, B), jnp.float32),
        compiler_params=pltpu.CompilerParams(vmem_limit_bytes=100 * 1024 * 1024),
    )(g_arr, Qh, Ql, Kh, Kl, *fw)

    delays, weights = pl.pallas_call(
        functools.partial(_stage2_body, L, B, K7, KP),
        in_specs=[pl.BlockSpec((L, B), lambda: (0, 0))],
        out_specs=[
            pl.BlockSpec((KP, B), lambda: (0, 0)),
            pl.BlockSpec((KP, B), lambda: (0, 0)),
        ],
        out_shape=[
            jax.ShapeDtypeStruct((KP, B), jnp.int32),
            jax.ShapeDtypeStruct((KP, B), jnp.float32),
        ],
    )(mv)

    out = pl.pallas_call(
        functools.partial(_stage3_body, L, K7),
        grid=(B, H),
        in_specs=[
            pl.BlockSpec((KP, B), lambda b, h: (0, 0), memory_space=pltpu.SMEM),
            pl.BlockSpec((KP, B), lambda b, h: (0, 0), memory_space=pltpu.SMEM),
            pl.BlockSpec((1, 1, L, E), lambda b, h: (b, h, 0, 0)),
        ],
        out_specs=pl.BlockSpec((1, 1, L, E), lambda b, h: (b, h, 0, 0)),
        out_shape=jax.ShapeDtypeStruct((B, H, L, E), jnp.float32),
        scratch_shapes=[pltpu.VMEM((2 * L, E), jnp.float32)],
    )(delays, weights, V)

    return out
